# baseline (speedup 1.0000x reference)
import functools

import jax
import jax.numpy as jnp
from jax import lax
from jax.experimental import pallas as pl
from jax.experimental.pallas import tpu as pltpu
from jax.experimental.pallas import tpu_sc as plsc

_LANES = 16
_UNROLL = 8


def kernel(input, embedding):
    idx = input.astype(jnp.int32)
    B, H = idx.shape
    V, E = embedding.shape
    info = plsc.get_sparse_core_info()
    NC, NS = info.num_cores, info.num_subcores
    NW = NC * NS
    BPW = B // NW
    EC = E // _LANES

    mesh = plsc.VectorSubcoreMesh(core_axis_name="c", subcore_axis_name="s")

    @functools.partial(
        pl.kernel,
        out_type=jax.ShapeDtypeStruct((B, E), jnp.float32),
        mesh=mesh,
        scratch_types=[
            pltpu.VMEM((H,), jnp.int32),
            pltpu.VMEM((H,), jnp.int32),
            pltpu.VMEM((H, E), jnp.float32),
            pltpu.VMEM((H, E), jnp.float32),
            pltpu.VMEM((BPW, E), jnp.float32),
            pltpu.SemaphoreType.DMA,
            pltpu.SemaphoreType.DMA,
            pltpu.SemaphoreType.DMA,
            pltpu.SemaphoreType.DMA,
        ],
        compiler_params=pltpu.CompilerParams(use_tc_tiling_on_sc=False),
    )
    def run(inp_hbm, tab_hbm, out_hbm, idx0, idx1, buf0, buf1, out_v,
            sem_i0, sem_i1, sem_g0, sem_g1):
        wid = lax.axis_index("s") * NC + lax.axis_index("c")
        base = wid * BPW
        idx_ref = (idx0, idx1)
        buf_ref = (buf0, buf1)
        sem_i = (sem_i0, sem_i1)
        sem_g = (sem_g0, sem_g1)

        def start_idx(r, slot):
            pltpu.async_copy(inp_hbm.at[base + r], idx_ref[slot], sem_i[slot])

        def wait_idx(r, slot):
            pltpu.make_async_copy(
                inp_hbm.at[base + r], idx_ref[slot], sem_i[slot]
            ).wait()

        def start_gather(slot):
            pltpu.async_copy(
                tab_hbm.at[idx_ref[slot]], buf_ref[slot], sem_g[slot]
            )

        def wait_gather(slot):
            pltpu.make_async_copy(
                tab_hbm.at[idx_ref[slot]], buf_ref[slot], sem_g[slot]
            ).wait()

        def reduce_row(buf, r):
            neg = jnp.full((_LANES,), -jnp.inf, dtype=jnp.float32)

            def inner(g, accs):
                accs = list(accs)
                for u in range(_UNROLL):
                    i = g * _UNROLL + u
                    for c in range(EC):
                        accs[c] = jnp.maximum(
                            accs[c], buf[i, pl.ds(c * _LANES, _LANES)]
                        )
                return tuple(accs)

            accs = lax.fori_loop(0, H // _UNROLL, inner, (neg,) * EC)
            rem = H % _UNROLL
            if rem:
                accs = list(accs)
                for u in range(rem):
                    i = H - rem + u
                    for c in range(EC):
                        accs[c] = jnp.maximum(
                            accs[c], buf[i, pl.ds(c * _LANES, _LANES)]
                        )
            for c in range(EC):
                x = accs[c]
                e = jnp.exp(-2.0 * jnp.abs(x))
                t = (1.0 - e) / (1.0 + e)
                out_v[r, pl.ds(c * _LANES, _LANES)] = jnp.where(x < 0.0, -t, t)

        start_idx(0, 0)
        start_idx(1, 1)
        wait_idx(0, 0)
        start_gather(0)

        def pair_body(g, carry):
            r0 = 2 * g
            r1 = r0 + 1
            wait_idx(r1, 1)
            start_gather(1)
            wait_gather(0)

            @pl.when(r1 + 1 < BPW)
            def _():
                start_idx(r1 + 1, 0)

            reduce_row(buf0, r0)

            @pl.when(r1 + 1 < BPW)
            def _():
                wait_idx(r1 + 1, 0)
                start_gather(0)

            wait_gather(1)

            @pl.when(r1 + 2 < BPW)
            def _():
                start_idx(r1 + 2, 1)

            reduce_row(buf1, r1)
            return carry

        lax.fori_loop(0, BPW // 2, pair_body, 0)
        pltpu.sync_copy(out_v, out_hbm.at[pl.ds(base, BPW)])

    return run(idx, embedding)


# speedup vs baseline: 2.2532x; 2.2532x over previous
import functools

import jax
import jax.numpy as jnp
from jax import lax
from jax.experimental import pallas as pl
from jax.experimental.pallas import tpu as pltpu
from jax.experimental.pallas import tpu_sc as plsc

_LANES = 16
_UNROLL = 8
_NBUF = 8
_REPACK_CW = 16384


def _split_point(V):
    nblk = -(-(V // 2) // _REPACK_CW)
    return nblk * _REPACK_CW


def _repack(embT):
    E, V = embT.shape
    CW = _REPACK_CW
    K = _split_point(V)
    grid = K // CW

    def body(x_ref, x2_ref, o_ref):
        y = jnp.transpose(x_ref[...])
        y2 = jnp.transpose(x2_ref[...])
        o_ref[...] = jnp.concatenate([y, y2], axis=1)

    koff = K // CW
    lastblk = pl.cdiv(V, CW) - 1
    return pl.pallas_call(
        body,
        grid=(grid,),
        in_specs=[
            pl.BlockSpec((E, CW), lambda i: (0, i)),
            pl.BlockSpec((E, CW), lambda i: (0, jnp.minimum(i + koff, lastblk))),
        ],
        out_specs=pl.BlockSpec((CW, 2 * E), lambda i: (i, 0)),
        out_shape=jax.ShapeDtypeStruct((K, 2 * E), jnp.float32),
    )(embT, embT)


def kernel(input, embedding):
    idx = input.astype(jnp.int32)
    B, H = idx.shape
    V, E = embedding.shape
    K = _split_point(V)
    repacked = _repack(embedding.T)
    repacked = jax.lax.optimization_barrier(repacked)
    embedding = repacked.reshape(2 * K, E)
    idx = jnp.where(idx < K, 2 * idx, 2 * (idx - K) + 1)
    info = plsc.get_sparse_core_info()
    NC, NS = info.num_cores, info.num_subcores
    NW = NC * NS
    BPW = B // NW
    EC = E // _LANES

    mesh = plsc.VectorSubcoreMesh(core_axis_name="c", subcore_axis_name="s")

    @functools.partial(
        pl.kernel,
        out_type=jax.ShapeDtypeStruct((B, E), jnp.float32),
        mesh=mesh,
        scratch_types=(
            [pltpu.VMEM((H,), jnp.int32)] * _NBUF
            + [pltpu.VMEM((H, E), jnp.float32)] * _NBUF
            + [pltpu.VMEM((BPW, E), jnp.float32)]
            + [pltpu.SemaphoreType.DMA] * (2 * _NBUF)
        ),
        compiler_params=pltpu.CompilerParams(use_tc_tiling_on_sc=False),
    )
    def run(inp_hbm, tab_hbm, out_hbm, *scr):
        wid = lax.axis_index("s") * NC + lax.axis_index("c")
        base = wid * BPW
        idx_ref = scr[0:_NBUF]
        buf_ref = scr[_NBUF:2 * _NBUF]
        out_v = scr[2 * _NBUF]
        sem_i = scr[2 * _NBUF + 1:3 * _NBUF + 1]
        sem_g = scr[3 * _NBUF + 1:4 * _NBUF + 1]

        def start_idx(r, slot):
            pltpu.async_copy(inp_hbm.at[base + r], idx_ref[slot], sem_i[slot])

        def wait_idx(r, slot):
            pltpu.make_async_copy(
                inp_hbm.at[base + r], idx_ref[slot], sem_i[slot]
            ).wait()

        def start_gather(slot):
            pltpu.async_copy(
                tab_hbm.at[idx_ref[slot]], buf_ref[slot], sem_g[slot]
            )

        def wait_gather(slot):
            pltpu.make_async_copy(
                tab_hbm.at[idx_ref[slot]], buf_ref[slot], sem_g[slot]
            ).wait()

        def reduce_row(buf, r):
            neg = jnp.full((_LANES,), -jnp.inf, dtype=jnp.float32)

            def inner(g, accs):
                accs = list(accs)
                for u in range(_UNROLL):
                    i = g * _UNROLL + u
                    for c in range(EC):
                        accs[c] = jnp.maximum(
                            accs[c], buf[i, pl.ds(c * _LANES, _LANES)]
                        )
                return tuple(accs)

            accs = lax.fori_loop(0, H // _UNROLL, inner, (neg,) * EC)
            rem = H % _UNROLL
            if rem:
                accs = list(accs)
                for u in range(rem):
                    i = H - rem + u
                    for c in range(EC):
                        accs[c] = jnp.maximum(
                            accs[c], buf[i, pl.ds(c * _LANES, _LANES)]
                        )
            for c in range(EC):
                x = accs[c]
                e = jnp.exp(-2.0 * jnp.abs(x))
                t = (1.0 - e) / (1.0 + e)
                out_v[r, pl.ds(c * _LANES, _LANES)] = jnp.where(x < 0.0, -t, t)

        for s in range(_NBUF):
            start_idx(s, s)
        for s in range(_NBUF - 1):
            wait_idx(s, s)
            start_gather(s)

        def group_body(g, carry):
            for u in range(_NBUF):
                r = _NBUF * g + u
                wait_gather(u)

                @pl.when(r + _NBUF < BPW)
                def _():
                    start_idx(r + _NBUF, u)

                @pl.when(r + _NBUF - 1 < BPW)
                def _():
                    wait_idx(r + _NBUF - 1, (u + _NBUF - 1) % _NBUF)
                    start_gather((u + _NBUF - 1) % _NBUF)

                reduce_row(buf_ref[u], r)
            return carry

        lax.fori_loop(0, BPW // _NBUF, group_body, 0)
        pltpu.sync_copy(out_v, out_hbm.at[pl.ds(base, BPW)])

    return run(idx, embedding)


# speedup vs baseline: 2.2674x; 1.0063x over previous
import functools

import jax
import jax.numpy as jnp
from jax import lax
from jax.experimental import pallas as pl
from jax.experimental.pallas import tpu as pltpu
from jax.experimental.pallas import tpu_sc as plsc

_LANES = 16
_UNROLL = 8
_REPACK_CW = 16384


def _split_point(V):
    nblk = -(-(V // 2) // _REPACK_CW)
    return nblk * _REPACK_CW


def _repack(embT):
    E, V = embT.shape
    CW = _REPACK_CW
    K = _split_point(V)
    grid = K // CW

    def body(x_ref, x2_ref, o_ref):
        y = jnp.transpose(x_ref[...])
        y2 = jnp.transpose(x2_ref[...])
        o_ref[...] = jnp.concatenate([y, y2], axis=1)

    koff = K // CW
    lastblk = pl.cdiv(V, CW) - 1
    return pl.pallas_call(
        body,
        grid=(grid,),
        in_specs=[
            pl.BlockSpec((E, CW), lambda i: (0, i)),
            pl.BlockSpec((E, CW), lambda i: (0, jnp.minimum(i + koff, lastblk))),
        ],
        out_specs=pl.BlockSpec((CW, 2 * E), lambda i: (i, 0)),
        out_shape=jax.ShapeDtypeStruct((K, 2 * E), jnp.float32),
    )(embT, embT)


def kernel(input, embedding):
    idx = input.astype(jnp.int32)
    B, H = idx.shape
    V, E = embedding.shape
    K = _split_point(V)
    repacked = _repack(embedding.T)
    repacked = jax.lax.optimization_barrier(repacked)
    embedding = repacked.reshape(2 * K, E)
    idx = jnp.where(idx < K, 2 * idx, 2 * (idx - K) + 1)
    info = plsc.get_sparse_core_info()
    NC, NS = info.num_cores, info.num_subcores
    NW = NC * NS
    BPW = B // NW
    EC = E // _LANES

    mesh = plsc.VectorSubcoreMesh(core_axis_name="c", subcore_axis_name="s")

    @functools.partial(
        pl.kernel,
        out_type=jax.ShapeDtypeStruct((B, E), jnp.float32),
        mesh=mesh,
        scratch_types=[
            pltpu.VMEM((H,), jnp.int32),
            pltpu.VMEM((H,), jnp.int32),
            pltpu.VMEM((H,), jnp.int32),
            pltpu.VMEM((H,), jnp.int32),
            pltpu.VMEM((H, E), jnp.float32),
            pltpu.VMEM((H, E), jnp.float32),
            pltpu.VMEM((H, E), jnp.float32),
            pltpu.VMEM((H, E), jnp.float32),
            pltpu.VMEM((BPW, E), jnp.float32),
            pltpu.SemaphoreType.DMA,
            pltpu.SemaphoreType.DMA,
            pltpu.SemaphoreType.DMA,
            pltpu.SemaphoreType.DMA,
            pltpu.SemaphoreType.DMA,
            pltpu.SemaphoreType.DMA,
            pltpu.SemaphoreType.DMA,
            pltpu.SemaphoreType.DMA,
        ],
        compiler_params=pltpu.CompilerParams(use_tc_tiling_on_sc=False),
    )
    def run(inp_hbm, tab_hbm, out_hbm, idx0, idx1, idx2, idx3,
            buf0, buf1, buf2, buf3, out_v,
            sem_i0, sem_i1, sem_i2, sem_i3,
            sem_g0, sem_g1, sem_g2, sem_g3):
        wid = lax.axis_index("s") * NC + lax.axis_index("c")
        base = wid * BPW
        idx_ref = (idx0, idx1, idx2, idx3)
        buf_ref = (buf0, buf1, buf2, buf3)
        sem_i = (sem_i0, sem_i1, sem_i2, sem_i3)
        sem_g = (sem_g0, sem_g1, sem_g2, sem_g3)

        def start_idx(r, slot):
            pltpu.async_copy(inp_hbm.at[base + r], idx_ref[slot], sem_i[slot])

        def wait_idx(r, slot):
            pltpu.make_async_copy(
                inp_hbm.at[base + r], idx_ref[slot], sem_i[slot]
            ).wait()

        def start_gather(slot):
            pltpu.async_copy(
                tab_hbm.at[idx_ref[slot]], buf_ref[slot], sem_g[slot]
            )

        def wait_gather(slot):
            pltpu.make_async_copy(
                tab_hbm.at[idx_ref[slot]], buf_ref[slot], sem_g[slot]
            ).wait()

        def reduce_row(buf, r):
            neg = jnp.full((_LANES,), -jnp.inf, dtype=jnp.float32)

            def inner(g, accs):
                accs = list(accs)
                for u in range(_UNROLL):
                    i = g * _UNROLL + u
                    for c in range(EC):
                        accs[c] = jnp.maximum(
                            accs[c], buf[i, pl.ds(c * _LANES, _LANES)]
                        )
                return tuple(accs)

            accs = lax.fori_loop(0, H // _UNROLL, inner, (neg,) * EC)
            rem = H % _UNROLL
            if rem:
                accs = list(accs)
                for u in range(rem):
                    i = H - rem + u
                    for c in range(EC):
                        accs[c] = jnp.maximum(
                            accs[c], buf[i, pl.ds(c * _LANES, _LANES)]
                        )
            for c in range(EC):
                x = accs[c]
                e = jnp.exp(-2.0 * jnp.abs(x))
                t = (1.0 - e) / (1.0 + e)
                out_v[r, pl.ds(c * _LANES, _LANES)] = jnp.where(x < 0.0, -t, t)

        for s in range(4):
            start_idx(s, s)
        for s in range(3):
            wait_idx(s, s)
            start_gather(s)

        def quad_body(g, carry):
            for u in range(4):
                r = 4 * g + u
                wait_gather(u)

                @pl.when(r + 4 < BPW)
                def _():
                    start_idx(r + 4, u)

                @pl.when(r + 3 < BPW)
                def _():
                    wait_idx(r + 3, (u + 3) % 4)
                    start_gather((u + 3) % 4)

                reduce_row(buf_ref[u], r)
            return carry

        lax.fori_loop(0, BPW // 4, quad_body, 0)
        pltpu.sync_copy(out_v, out_hbm.at[pl.ds(base, BPW)])

    return run(idx, embedding)
